# baseline (speedup 1.0000x reference)
import functools

import jax
import jax.numpy as jnp
from jax.experimental import pallas as pl

_NUM_DIFF_LAYERS = 10
_C = 0.15
_BM = 400
_BM_FIRST = 200

_A_SCALE = 1024.0
_LO_SCALE = 64.0
_F8 = jnp.float8_e4m3fn


def _steady_step_kernel(ap_ref, am_ref, hi_ref, lo_ref, tx_ref, *out_refs,
                        d, last):
    ap = ap_ref[...]
    am = am_ref[...]
    hi = hi_ref[...]
    lo = lo_ref[...]
    dn = (((1,), (0,)), ((), ()))
    dot = functools.partial(jax.lax.dot_general, dimension_numbers=dn,
                            preferred_element_type=jnp.float32)
    inv = 1.0 / _A_SCALE
    inv_lo = 1.0 / (_A_SCALE * _LO_SCALE)
    y1 = dot(ap, hi) * inv + dot(ap, lo) * inv_lo
    y2 = dot(am, hi) * inv + dot(am, lo) * inv_lo
    newp = y1[:, :d] + y2[:, d:] + tx_ref[...]
    newm = y2[:, :d] + y1[:, d:]
    out = jnp.concatenate([newp, newm], axis=1)
    if last:
        out_refs[0][...] = out
    else:
        new_hi = out.astype(_F8)
        out_refs[0][...] = new_hi
        out_refs[1][...] = ((out - new_hi.astype(jnp.float32))
                            * _LO_SCALE).astype(_F8)


def _steady_step(ap8, am8, hi, lo, tx, bm, last):
    n = hi.shape[0]
    d = tx.shape[1]
    blk = pl.BlockSpec((bm, 2 * d), lambda i: (i, 0))
    return pl.pallas_call(
        functools.partial(_steady_step_kernel, d=d, last=last),
        grid=(n // bm,),
        in_specs=[
            pl.BlockSpec((bm, n), lambda i: (i, 0)),
            pl.BlockSpec((bm, n), lambda i: (i, 0)),
            pl.BlockSpec((n, 2 * d), lambda i: (0, 0)),
            pl.BlockSpec((n, 2 * d), lambda i: (0, 0)),
            pl.BlockSpec((bm, d), lambda i: (i, 0)),
        ],
        out_specs=[blk] if last else [blk, blk],
        out_shape=(
            [jax.ShapeDtypeStruct((n, 2 * d), jnp.float32)] if last else
            [jax.ShapeDtypeStruct((n, 2 * d), _F8),
             jax.ShapeDtypeStruct((n, 2 * d), _F8)]
        ),
    )(ap8, am8, hi, lo, tx)


def _first_step_kernel(ap_ref, am_ref, pm_ref, tx_ref,
                       hi_ref, lo_ref, ap8_ref, am8_ref, *, d):
    ap32 = ap_ref[...]
    am32 = am_ref[...]
    ap8_ref[...] = (ap32 * _A_SCALE).astype(_F8)
    am8_ref[...] = (am32 * _A_SCALE).astype(_F8)
    ap = ap32.astype(jnp.bfloat16)
    am = am32.astype(jnp.bfloat16)
    pm = pm_ref[...]
    y1 = jnp.dot(ap, pm, preferred_element_type=jnp.float32)
    y2 = jnp.dot(am, pm, preferred_element_type=jnp.float32)
    newp = y1[:, :d] + y2[:, d:] + tx_ref[...]
    newm = y2[:, :d] + y1[:, d:]
    out = jnp.concatenate([newp, newm], axis=1)
    new_hi = out.astype(_F8)
    hi_ref[...] = new_hi
    lo_ref[...] = ((out - new_hi.astype(jnp.float32))
                   * _LO_SCALE).astype(_F8)


def _first_step(ap, am, pm, tx, bm):
    n = pm.shape[0]
    d = tx.shape[1]
    blk = pl.BlockSpec((bm, 2 * d), lambda i: (i, 0))
    return pl.pallas_call(
        functools.partial(_first_step_kernel, d=d),
        grid=(n // bm,),
        in_specs=[
            pl.BlockSpec((bm, n), lambda i: (i, 0)),
            pl.BlockSpec((bm, n), lambda i: (i, 0)),
            pl.BlockSpec((n, 2 * d), lambda i: (0, 0)),
            pl.BlockSpec((bm, d), lambda i: (i, 0)),
        ],
        out_specs=[
            blk,
            blk,
            pl.BlockSpec((bm, n), lambda i: (i, 0)),
            pl.BlockSpec((bm, n), lambda i: (i, 0)),
        ],
        out_shape=[
            jax.ShapeDtypeStruct((n, 2 * d), _F8),
            jax.ShapeDtypeStruct((n, 2 * d), _F8),
            jax.ShapeDtypeStruct((n, n), _F8),
            jax.ShapeDtypeStruct((n, n), _F8),
        ],
    )(ap, am, pm, tx)


def kernel(nApT, nAmT, X):
    m0 = jax.random.uniform(jax.random.key(1), X.shape, dtype=jnp.float32,
                            minval=-1.0, maxval=1.0)
    tx = _C * X
    pm0 = jnp.concatenate([X, m0], axis=1).astype(jnp.bfloat16)
    hi, lo, ap8, am8 = _first_step(nApT, nAmT, pm0, tx, _BM_FIRST)
    for layer in range(1, _NUM_DIFF_LAYERS - 1):
        hi, lo = _steady_step(ap8, am8, hi, lo, tx, _BM, last=False)
    (pm,) = _steady_step(ap8, am8, hi, lo, tx, _BM, last=True)
    d = X.shape[1]
    return (pm[:, :d], pm[:, d:])


# speedup vs baseline: 1.0941x; 1.0941x over previous
import functools

import jax
import jax.numpy as jnp
from jax import lax
from jax.experimental import pallas as pl
from jax.experimental.pallas import tpu as pltpu

_NUM_DIFF_LAYERS = 10
_C = 0.15
_BM = 400
_BM_FIRST = 200

_A_SCALE = 1024.0
_F8 = jnp.float8_e4m3fn
_DN = (((1,), (0,)), ((), ()))


def _diffusion_kernel(ap_ref, am_ref, pm0_ref, tx_ref, p_ref, m_ref,
                      s0_ref, s1_ref, *, d, bm, nsteps):
    s = pl.program_id(0)
    i = pl.program_id(1)

    @pl.when(jnp.logical_and(s == 0, i == 0))
    def _():
        s0_ref[...] = pm0_ref[...]

    def body(cur_ref, nxt_ref):
        ap = ap_ref[...].astype(jnp.bfloat16)
        am = am_ref[...].astype(jnp.bfloat16)
        pm = cur_ref[...]
        y1 = lax.dot_general(ap, pm, _DN,
                             preferred_element_type=jnp.float32)
        y2 = lax.dot_general(am, pm, _DN,
                             preferred_element_type=jnp.float32)
        inv = 1.0 / _A_SCALE
        newp = (y1[:, :d] + y2[:, d:]) * inv + tx_ref[...]
        newm = (y2[:, :d] + y1[:, d:]) * inv
        p_ref[...] = newp
        m_ref[...] = newm
        out16 = jnp.concatenate([newp, newm], axis=1).astype(jnp.bfloat16)
        nxt_ref[pl.ds(i * bm, bm), :] = out16

    @pl.when(lax.rem(s, 2) == 0)
    def _():
        body(s0_ref, s1_ref)

    @pl.when(lax.rem(s, 2) == 1)
    def _():
        body(s1_ref, s0_ref)


def _diffusion(ap8, am8, pm0, tx, bm, nsteps):
    n = pm0.shape[0]
    d = tx.shape[1]
    return pl.pallas_call(
        functools.partial(_diffusion_kernel, d=d, bm=bm, nsteps=nsteps),
        grid=(nsteps, n // bm),
        in_specs=[
            pl.BlockSpec((bm, n), lambda s, i: (i, 0)),
            pl.BlockSpec((bm, n), lambda s, i: (i, 0)),
            pl.BlockSpec((n, 2 * d), lambda s, i: (0, 0)),
            pl.BlockSpec((bm, d), lambda s, i: (i, 0)),
        ],
        out_specs=[
            pl.BlockSpec((bm, d), lambda s, i: (i, 0)),
            pl.BlockSpec((bm, d), lambda s, i: (i, 0)),
        ],
        out_shape=[
            jax.ShapeDtypeStruct((n, d), jnp.float32),
            jax.ShapeDtypeStruct((n, d), jnp.float32),
        ],
        scratch_shapes=[
            pltpu.VMEM((n, 2 * d), jnp.bfloat16),
            pltpu.VMEM((n, 2 * d), jnp.bfloat16),
        ],
    )(ap8, am8, pm0, tx)


def _first_step_kernel(ap_ref, am_ref, pm_ref, tx_ref,
                       out_ref, ap8_ref, am8_ref, *, d):
    ap32 = ap_ref[...]
    am32 = am_ref[...]
    ap8_ref[...] = (ap32 * _A_SCALE).astype(_F8)
    am8_ref[...] = (am32 * _A_SCALE).astype(_F8)
    ap = ap32.astype(jnp.bfloat16)
    am = am32.astype(jnp.bfloat16)
    pm = pm_ref[...]
    y1 = jnp.dot(ap, pm, preferred_element_type=jnp.float32)
    y2 = jnp.dot(am, pm, preferred_element_type=jnp.float32)
    newp = y1[:, :d] + y2[:, d:] + tx_ref[...]
    newm = y2[:, :d] + y1[:, d:]
    out_ref[...] = jnp.concatenate([newp, newm], axis=1).astype(jnp.bfloat16)


def _first_step(ap, am, pm, tx, bm):
    n = pm.shape[0]
    d = tx.shape[1]
    return pl.pallas_call(
        functools.partial(_first_step_kernel, d=d),
        grid=(n // bm,),
        in_specs=[
            pl.BlockSpec((bm, n), lambda i: (i, 0)),
            pl.BlockSpec((bm, n), lambda i: (i, 0)),
            pl.BlockSpec((n, 2 * d), lambda i: (0, 0)),
            pl.BlockSpec((bm, d), lambda i: (i, 0)),
        ],
        out_specs=[
            pl.BlockSpec((bm, 2 * d), lambda i: (i, 0)),
            pl.BlockSpec((bm, n), lambda i: (i, 0)),
            pl.BlockSpec((bm, n), lambda i: (i, 0)),
        ],
        out_shape=[
            jax.ShapeDtypeStruct((n, 2 * d), jnp.bfloat16),
            jax.ShapeDtypeStruct((n, n), _F8),
            jax.ShapeDtypeStruct((n, n), _F8),
        ],
    )(ap, am, pm, tx)


def kernel(nApT, nAmT, X):
    m0 = jax.random.uniform(jax.random.key(1), X.shape, dtype=jnp.float32,
                            minval=-1.0, maxval=1.0)
    tx = _C * X
    pm0 = jnp.concatenate([X, m0], axis=1).astype(jnp.bfloat16)
    pm1, ap8, am8 = _first_step(nApT, nAmT, pm0, tx, _BM_FIRST)
    return _diffusion(ap8, am8, pm1, tx, _BM, _NUM_DIFF_LAYERS - 1)
